# Initial kernel scaffold; baseline (speedup 1.0000x reference)
import jax
import jax.numpy as jnp
from jax.experimental import pallas as pl


def kernel(nodes, edges, senders, receivers):
    raise NotImplementedError("write your pallas kernel here")


# speedup vs baseline: 7.8334x; 7.8334x over previous
import functools

import jax
import jax.numpy as jnp
from jax import lax
from jax.experimental import pallas as pl
from jax.experimental.pallas import tpu as pltpu
from jax.experimental.pallas import tpu_sc as plsc

NC = 2
NS = 16
NW = NC * NS

STREAM = 128
K = 4
CHUNK = K * STREAM


def _sc_scatter(edges, sidx, ridx, zeros):
    e, d = edges.shape
    n = zeros.shape[0]
    total_chunks = e // CHUNK
    iters = (total_chunks + NW - 1) // NW
    rblk = 1000
    nblk = n // rblk
    blk_iters = (nblk + NS - 1) // NS

    mesh = plsc.VectorSubcoreMesh(
        core_axis_name="c", subcore_axis_name="s", num_cores=NC, num_subcores=NS
    )

    @functools.partial(
        pl.kernel,
        mesh=mesh,
        out_type=jax.ShapeDtypeStruct((NC * n, d), jnp.float32),
        scratch_types=[
            pltpu.VMEM((CHUNK, d), jnp.float32),
            pltpu.VMEM((K, STREAM), jnp.int32),
            pltpu.VMEM((K, STREAM), jnp.int32),
            pltpu.VMEM_SHARED((n, d), jnp.float32),
        ],
        compiler_params=pltpu.CompilerParams(use_tc_tiling_on_sc=False),
    )
    def run(edges_hbm, sidx_hbm, ridx_hbm, zeros_hbm, out_hbm, ebuf, sbuf, rbuf, acc):
        c = lax.axis_index("c")
        s = lax.axis_index("s")
        wid = c * NS + s

        def zero_body(i, carry):
            blk = s + NS * i

            @pl.when(blk < nblk)
            def _():
                pltpu.sync_copy(
                    zeros_hbm.at[pl.ds(blk * rblk, rblk)],
                    acc.at[pl.ds(blk * rblk, rblk)],
                )

            return carry

        lax.fori_loop(0, blk_iters, zero_body, None)
        plsc.subcore_barrier()

        def chunk_body(i, carry):
            chunk = wid + NW * i

            @pl.when(chunk < total_chunks)
            def _():
                pltpu.sync_copy(edges_hbm.at[pl.ds(chunk * CHUNK, CHUNK)], ebuf)
                pltpu.sync_copy(sidx_hbm.at[pl.ds(chunk * K, K)], sbuf)
                pltpu.sync_copy(ridx_hbm.at[pl.ds(chunk * K, K)], rbuf)
                for j in range(K):
                    rows = ebuf.at[pl.ds(j * STREAM, STREAM)]
                    pltpu.sync_copy(rows, acc.at[sbuf.at[j]], add=True)
                    pltpu.sync_copy(rows, acc.at[rbuf.at[j]], add=True)

            return carry

        lax.fori_loop(0, iters, chunk_body, None)

        plsc.subcore_barrier()

        def out_body(i, carry):
            blk = s + NS * i

            @pl.when(blk < nblk)
            def _():
                pltpu.sync_copy(
                    acc.at[pl.ds(blk * rblk, rblk)],
                    out_hbm.at[pl.ds(c * n + blk * rblk, rblk)],
                )

            return carry

        lax.fori_loop(0, blk_iters, out_body, None)

    return run(edges, sidx, ridx, zeros)


def _tc_combine(partials, n, d):
    blk = 1000
    grid = n // blk

    def body(a_ref, b_ref, o_ref):
        o_ref[...] = a_ref[...] + b_ref[...]

    return pl.pallas_call(
        body,
        out_shape=jax.ShapeDtypeStruct((n, d), jnp.float32),
        grid=(grid,),
        in_specs=[
            pl.BlockSpec((blk, d), lambda i: (i, 0)),
            pl.BlockSpec((blk, d), lambda i: (i + grid, 0)),
        ],
        out_specs=pl.BlockSpec((blk, d), lambda i: (i, 0)),
    )(partials, partials)


def kernel(nodes, edges, senders, receivers):
    n, d = nodes.shape
    e = edges.shape[0]
    sidx = senders.reshape(e // STREAM, STREAM)
    ridx = receivers.reshape(e // STREAM, STREAM)
    zeros = jnp.zeros((n, d), dtype=jnp.float32)
    partials = _sc_scatter(edges, sidx, ridx, zeros)
    return _tc_combine(partials, n, d)


# speedup vs baseline: 9.5112x; 1.2142x over previous
import functools

import jax
import jax.numpy as jnp
from jax import lax
from jax.experimental import pallas as pl
from jax.experimental.pallas import tpu as pltpu
from jax.experimental.pallas import tpu_sc as plsc

NC = 2
NS = 16
NW = NC * NS

STREAM = 128
K = 8
CHUNK = K * STREAM


def _sc_scatter(edges, sidx, ridx, zeros):
    e, d = edges.shape
    n = zeros.shape[0]
    total_chunks = e // CHUNK
    iters = (total_chunks + NW - 1) // NW
    rblk = 1000
    nblk = n // rblk
    blk_iters = (nblk + NS - 1) // NS

    mesh = plsc.VectorSubcoreMesh(
        core_axis_name="c", subcore_axis_name="s", num_cores=NC, num_subcores=NS
    )

    @functools.partial(
        pl.kernel,
        mesh=mesh,
        out_type=jax.ShapeDtypeStruct((NC * n, d), jnp.float32),
        scratch_types=[
            pltpu.VMEM((CHUNK, d), jnp.float32),
            pltpu.VMEM((K, STREAM), jnp.int32),
            pltpu.VMEM((K, STREAM), jnp.int32),
            pltpu.VMEM_SHARED((n, d), jnp.float32),
            pltpu.SemaphoreType.DMA,
            pltpu.SemaphoreType.DMA,
        ],
        compiler_params=pltpu.CompilerParams(use_tc_tiling_on_sc=False),
    )
    def run(edges_hbm, sidx_hbm, ridx_hbm, zeros_hbm, out_hbm, ebuf, sbuf, rbuf,
            acc, fetch_sem, scat_sem):
        c = lax.axis_index("c")
        s = lax.axis_index("s")
        wid = c * NS + s

        def zero_body(i, carry):
            blk = s + NS * i

            @pl.when(blk < nblk)
            def _():
                pltpu.sync_copy(
                    zeros_hbm.at[pl.ds(blk * rblk, rblk)],
                    acc.at[pl.ds(blk * rblk, rblk)],
                )

            return carry

        lax.fori_loop(0, blk_iters, zero_body, None)
        plsc.subcore_barrier()

        def chunk_body(i, carry):
            chunk = wid + NW * i

            @pl.when(chunk < total_chunks)
            def _():
                fetches = [
                    pltpu.async_copy(
                        edges_hbm.at[pl.ds(chunk * CHUNK, CHUNK)], ebuf, fetch_sem
                    ),
                    pltpu.async_copy(
                        sidx_hbm.at[pl.ds(chunk * K, K)], sbuf, fetch_sem
                    ),
                    pltpu.async_copy(
                        ridx_hbm.at[pl.ds(chunk * K, K)], rbuf, fetch_sem
                    ),
                ]
                for f in fetches:
                    f.wait()
                scats = []
                for j in range(K):
                    rows = ebuf.at[pl.ds(j * STREAM, STREAM)]
                    scats.append(
                        pltpu.async_copy(rows, acc.at[sbuf.at[j]], scat_sem, add=True)
                    )
                    scats.append(
                        pltpu.async_copy(rows, acc.at[rbuf.at[j]], scat_sem, add=True)
                    )
                for sc in scats:
                    sc.wait()

            return carry

        lax.fori_loop(0, iters, chunk_body, None)

        plsc.subcore_barrier()

        def out_body(i, carry):
            blk = s + NS * i

            @pl.when(blk < nblk)
            def _():
                pltpu.sync_copy(
                    acc.at[pl.ds(blk * rblk, rblk)],
                    out_hbm.at[pl.ds(c * n + blk * rblk, rblk)],
                )

            return carry

        lax.fori_loop(0, blk_iters, out_body, None)

    return run(edges, sidx, ridx, zeros)


def _tc_combine(partials, n, d):
    blk = 1000
    grid = n // blk

    def body(a_ref, b_ref, o_ref):
        o_ref[...] = a_ref[...] + b_ref[...]

    return pl.pallas_call(
        body,
        out_shape=jax.ShapeDtypeStruct((n, d), jnp.float32),
        grid=(grid,),
        in_specs=[
            pl.BlockSpec((blk, d), lambda i: (i, 0)),
            pl.BlockSpec((blk, d), lambda i: (i + grid, 0)),
        ],
        out_specs=pl.BlockSpec((blk, d), lambda i: (i, 0)),
    )(partials, partials)


def kernel(nodes, edges, senders, receivers):
    n, d = nodes.shape
    e = edges.shape[0]
    sidx = senders.reshape(e // STREAM, STREAM)
    ridx = receivers.reshape(e // STREAM, STREAM)
    zeros = jnp.zeros((n, d), dtype=jnp.float32)
    partials = _sc_scatter(edges, sidx, ridx, zeros)
    return _tc_combine(partials, n, d)


# speedup vs baseline: 10.3522x; 1.0884x over previous
import functools

import jax
import jax.numpy as jnp
from jax import lax
from jax.experimental import pallas as pl
from jax.experimental.pallas import tpu as pltpu
from jax.experimental.pallas import tpu_sc as plsc

NC = 2
NS = 16
NW = NC * NS

GRP = 128
GB = 4
GW = GB * 8 * GRP


def _sc_scatter(x1, sidx, ridx, zeros):
    d = 16
    n = zeros.shape[0]
    n_groups = sidx.shape[0]
    half_words = x1.shape[0] // 2
    groups_per_tile = -(-n_groups // NW)
    groups_per_tile = -(-groups_per_tile // (2 * GB)) * (2 * GB)
    batches = groups_per_tile // GB
    rblk = 1000
    nblk = n // rblk
    blk_iters = (nblk + NS - 1) // NS

    mesh = plsc.VectorSubcoreMesh(
        core_axis_name="c", subcore_axis_name="s", num_cores=NC, num_subcores=NS
    )

    @functools.partial(
        pl.kernel,
        mesh=mesh,
        out_type=jax.ShapeDtypeStruct((NC * n, d), jnp.float32),
        scratch_types=[
            pltpu.VMEM((2 * 2 * GW,), jnp.float32),
            pltpu.VMEM((GB, GRP, d), jnp.float32),
            pltpu.VMEM((2, GB, GRP), jnp.int32),
            pltpu.VMEM((2, GB, GRP), jnp.int32),
            pltpu.VMEM_SHARED((n, d), jnp.float32),
            pltpu.SemaphoreType.DMA,
        ] + [pltpu.SemaphoreType.DMA] * GB,
        compiler_params=pltpu.CompilerParams(
            use_tc_tiling_on_sc=False, needs_layout_passes=False
        ),
    )
    def run(x_hbm, sidx_hbm, ridx_hbm, zeros_hbm, out_hbm, fbuf, ebuf, sbuf, rbuf,
            acc, fetch_sem, *ssems):
        c = lax.axis_index("c")
        s = lax.axis_index("s")
        wid = c * NS + s
        gstart = wid * groups_per_tile

        def zero_body(i, carry):
            blk = s + NS * i

            @pl.when(blk < nblk)
            def _():
                pltpu.sync_copy(
                    zeros_hbm.at[pl.ds(blk * rblk, rblk)],
                    acc.at[pl.ds(blk * rblk, rblk)],
                )

            return carry

        lax.fori_loop(0, blk_iters, zero_body, None)
        plsc.subcore_barrier()

        fiota = lax.iota(jnp.int32, 16)
        g0vec = (fiota // 8) * GW + (fiota % 8) * GRP

        def fetch_ops(b, pb):
            g0 = gstart + b * GB
            return [
                (x_hbm.at[pl.ds(g0 * (8 * GRP), GW)],
                 fbuf.at[pl.ds(pb * (2 * GW), GW)]),
                (x_hbm.at[pl.ds(half_words + g0 * (8 * GRP), GW)],
                 fbuf.at[pl.ds(pb * (2 * GW) + GW, GW)]),
                (sidx_hbm.at[pl.ds(g0, GB)], sbuf.at[pb]),
                (ridx_hbm.at[pl.ds(g0, GB)], rbuf.at[pb]),
            ]

        def fetch_now(b, pb):
            descs = [
                pltpu.async_copy(src, dst, fetch_sem)
                for src, dst in fetch_ops(b, pb)
            ]
            for dsc in descs:
                dsc.wait()

        def drain(slot):
            sem = ssems[slot]
            other = (slot + 1) % GB
            pltpu.make_async_copy(
                zeros_hbm.at[pl.ds(0, GRP)], ebuf.at[other], sem
            ).wait()
            pltpu.make_async_copy(
                zeros_hbm.at[pl.ds(0, GRP)], ebuf.at[other], sem
            ).wait()

        def super_body(bb, carry):
            for pb in range(2):
                b = 2 * bb + pb
                g0 = gstart + b * GB

                @pl.when(g0 < n_groups)
                def _(pb=pb, b=b, g0=g0):
                    fetch_now(b, pb)
                    base = pb * (2 * GW)
                    for j in range(GB):
                        @pl.when(g0 + j < n_groups)
                        def _(pb=pb, b=b, j=j):
                            @pl.when(b > 0)
                            def _():
                                drain(j)
                            gj = g0vec + (base + j * (8 * GRP))

                            def tbody(cc, carry2):
                                cvec = gj + cc * 8
                                for u in range(8):
                                    val = plsc.load_gather(fbuf, [cvec + u])
                                    ebuf[j, cc * 8 + u, :] = val
                                return carry2

                            lax.fori_loop(0, GRP // 8, tbody, None)
                            pltpu.async_copy(
                                ebuf.at[j], acc.at[sbuf.at[pb, j]], ssems[j],
                                add=True,
                            )
                            pltpu.async_copy(
                                ebuf.at[j], acc.at[rbuf.at[pb, j]], ssems[j],
                                add=True,
                            )

            return carry

        lax.fori_loop(0, batches // 2, super_body, None)
        for k in range(GB):
            drain(k)

        plsc.subcore_barrier()

        def out_body(i, carry):
            blk = s + NS * i

            @pl.when(blk < nblk)
            def _():
                pltpu.sync_copy(
                    acc.at[pl.ds(blk * rblk, rblk)],
                    out_hbm.at[pl.ds(c * n + blk * rblk, rblk)],
                )

            return carry

        lax.fori_loop(0, blk_iters, out_body, None)

    return run(x1, sidx, ridx, zeros)


def _tc_combine(p3):
    rows = p3.shape[1]

    def body(a_ref, b_ref, o_ref):
        o_ref[...] = a_ref[0] + b_ref[0]

    return pl.pallas_call(
        body,
        out_shape=jax.ShapeDtypeStruct((rows, 128), jnp.float32),
        grid=(1,),
        in_specs=[
            pl.BlockSpec((1, rows, 128), lambda i: (0, 0, 0)),
            pl.BlockSpec((1, rows, 128), lambda i: (1, 0, 0)),
        ],
        out_specs=pl.BlockSpec((rows, 128), lambda i: (0, 0)),
    )(p3, p3)


def kernel(nodes, edges, senders, receivers):
    n, d = nodes.shape
    e = edges.shape[0]
    x1 = (
        edges.T.reshape(2, 8, e // GRP, GRP)
        .transpose(0, 2, 1, 3)
        .reshape(e * d)
    )
    sidx = senders.reshape(e // GRP, GRP)
    ridx = receivers.reshape(e // GRP, GRP)
    zeros = jnp.zeros((n, d), dtype=jnp.float32)
    partials = _sc_scatter(x1, sidx, ridx, zeros)
    out = _tc_combine(partials.reshape(NC, n * d // 128, 128))
    return out.reshape(n, d)


# speedup vs baseline: 22.4668x; 2.1702x over previous
import functools

import jax
import jax.numpy as jnp
from jax import lax
from jax.experimental import pallas as pl
from jax.experimental.pallas import tpu as pltpu
from jax.experimental.pallas import tpu_sc as plsc

NC = 2
NS = 16
NW = NC * NS

GRP = 128
GB = 4
GW = GB * 8 * GRP


def _sc_scatter(x1, sidx, ridx, zeros):
    d = 16
    n = zeros.shape[0]
    n_groups = sidx.shape[0]
    half_words = x1.shape[0] // 2
    groups_per_tile = -(-n_groups // NW)
    groups_per_tile = -(-groups_per_tile // (2 * GB)) * (2 * GB)
    batches = groups_per_tile // GB
    rblk = 1000
    nblk = n // rblk
    blk_iters = (nblk + NS - 1) // NS

    mesh = plsc.VectorSubcoreMesh(
        core_axis_name="c", subcore_axis_name="s", num_cores=NC, num_subcores=NS
    )

    @functools.partial(
        pl.kernel,
        mesh=mesh,
        out_type=jax.ShapeDtypeStruct((NC * n, d), jnp.float32),
        scratch_types=[
            pltpu.VMEM((2 * 2 * GW,), jnp.float32),
            pltpu.VMEM((GB * GRP, d), jnp.float32),
            pltpu.VMEM((2, GB, GRP), jnp.int32),
            pltpu.VMEM((2, GB, GRP), jnp.int32),
            pltpu.VMEM_SHARED((n, d), jnp.float32),
            pltpu.SemaphoreType.DMA,
        ] + [pltpu.SemaphoreType.DMA] * GB,
        compiler_params=pltpu.CompilerParams(
            use_tc_tiling_on_sc=False, needs_layout_passes=False
        ),
    )
    def run(x_hbm, sidx_hbm, ridx_hbm, zeros_hbm, out_hbm, fbuf, ebuf, sbuf, rbuf,
            acc, fetch_sem, *ssems):
        c = lax.axis_index("c")
        s = lax.axis_index("s")
        wid = c * NS + s
        gstart = wid * groups_per_tile

        def zero_body(i, carry):
            blk = s + NS * i

            @pl.when(blk < nblk)
            def _():
                pltpu.sync_copy(
                    zeros_hbm.at[pl.ds(blk * rblk, rblk)],
                    acc.at[pl.ds(blk * rblk, rblk)],
                )

            return carry

        lax.fori_loop(0, blk_iters, zero_body, None)
        plsc.subcore_barrier()

        fiota = lax.iota(jnp.int32, 16)
        pvec = (fiota // 8) * GW + (fiota % 8) * GRP
        diags = [(fiota + k) % 16 for k in range(16)]
        gdiag = [pvec + dk for dk in diags]

        def fetch_ops(b, pb):
            g0 = gstart + b * GB
            return [
                (x_hbm.at[pl.ds(g0 * (8 * GRP), GW)],
                 fbuf.at[pl.ds(pb * (2 * GW), GW)]),
                (x_hbm.at[pl.ds(half_words + g0 * (8 * GRP), GW)],
                 fbuf.at[pl.ds(pb * (2 * GW) + GW, GW)]),
                (sidx_hbm.at[pl.ds(g0, GB)], sbuf.at[pb]),
                (ridx_hbm.at[pl.ds(g0, GB)], rbuf.at[pb]),
            ]

        def fetch_now(b, pb):
            descs = [
                pltpu.async_copy(src, dst, fetch_sem)
                for src, dst in fetch_ops(b, pb)
            ]
            for dsc in descs:
                dsc.wait()

        def drain(slot):
            sem = ssems[slot]
            other = ((slot + 1) % GB) * GRP
            pltpu.make_async_copy(
                zeros_hbm.at[pl.ds(0, GRP)], ebuf.at[pl.ds(other, GRP)], sem
            ).wait()
            pltpu.make_async_copy(
                zeros_hbm.at[pl.ds(0, GRP)], ebuf.at[pl.ds(other, GRP)], sem
            ).wait()

        def super_body(bb, carry):
            for pb in range(2):
                b = 2 * bb + pb
                g0 = gstart + b * GB

                @pl.when(g0 < n_groups)
                def _(pb=pb, b=b, g0=g0):
                    fetch_now(b, pb)
                    base = pb * (2 * GW)
                    for j in range(GB):
                        @pl.when(g0 + j < n_groups)
                        def _(pb=pb, b=b, j=j):
                            @pl.when(b > 0)
                            def _():
                                drain(j)
                            src0 = base + j * (8 * GRP)
                            dst0 = j * GRP

                            def tbody(cc, carry2):
                                c0 = cc * 16
                                for k in range(16):
                                    val = plsc.load_gather(
                                        fbuf, [gdiag[k] + (src0 + c0)]
                                    )
                                    plsc.store_scatter(
                                        ebuf,
                                        [diags[k] + (dst0 + c0), fiota],
                                        val,
                                    )
                                return carry2

                            lax.fori_loop(0, GRP // 16, tbody, None)
                            src_rows = ebuf.at[pl.ds(j * GRP, GRP)]
                            pltpu.async_copy(
                                src_rows, acc.at[sbuf.at[pb, j]], ssems[j],
                                add=True,
                            )
                            pltpu.async_copy(
                                src_rows, acc.at[rbuf.at[pb, j]], ssems[j],
                                add=True,
                            )

            return carry

        lax.fori_loop(0, batches // 2, super_body, None)
        for k in range(GB):
            drain(k)

        plsc.subcore_barrier()

        def out_body(i, carry):
            blk = s + NS * i

            @pl.when(blk < nblk)
            def _():
                pltpu.sync_copy(
                    acc.at[pl.ds(blk * rblk, rblk)],
                    out_hbm.at[pl.ds(c * n + blk * rblk, rblk)],
                )

            return carry

        lax.fori_loop(0, blk_iters, out_body, None)

    return run(x1, sidx, ridx, zeros)


def _tc_combine(p3):
    rows = p3.shape[1]

    def body(a_ref, b_ref, o_ref):
        o_ref[...] = a_ref[0] + b_ref[0]

    return pl.pallas_call(
        body,
        out_shape=jax.ShapeDtypeStruct((rows, 128), jnp.float32),
        grid=(1,),
        in_specs=[
            pl.BlockSpec((1, rows, 128), lambda i: (0, 0, 0)),
            pl.BlockSpec((1, rows, 128), lambda i: (1, 0, 0)),
        ],
        out_specs=pl.BlockSpec((rows, 128), lambda i: (0, 0)),
    )(p3, p3)


def kernel(nodes, edges, senders, receivers):
    n, d = nodes.shape
    e = edges.shape[0]
    x1 = (
        edges.T.reshape(2, 8, e // GRP, GRP)
        .transpose(0, 2, 1, 3)
        .reshape(e * d)
    )
    sidx = senders.reshape(e // GRP, GRP)
    ridx = receivers.reshape(e // GRP, GRP)
    zeros = jnp.zeros((n, d), dtype=jnp.float32)
    partials = _sc_scatter(x1, sidx, ridx, zeros)
    out = _tc_combine(partials.reshape(NC, n * d // 128, 128))
    return out.reshape(n, d)
